# baseline (speedup 1.0000x reference)
import functools

import jax
import jax.numpy as jnp
from jax import lax
from jax.experimental import pallas as pl
from jax.experimental.pallas import tpu as pltpu
from jax.experimental.pallas import tpu_sc as plsc

NC = 2
NS = 16
NW = NC * NS
H = 64
F32 = jnp.float32


def _silu(v):
    return v * jax.nn.sigmoid(v)


def _ln(v, g, b, eps=1e-5):
    mu = jnp.mean(v, axis=-1, keepdims=True)
    var = jnp.mean((v - mu) ** 2, axis=-1, keepdims=True)
    return (v - mu) / jnp.sqrt(var + eps) * g + b


def _sc_mesh():
    return plsc.VectorSubcoreMesh(
        core_axis_name="c", subcore_axis_name="s", num_cores=NC, num_subcores=NS
    )


def _sc_gather2(table, idx_a, idx_b):
    v, d = table.shape
    dt = table.dtype
    b = idx_a.shape[0]
    b_per_w = b // NW
    s_g = 1000 if d == 64 else 2000
    n_ch = b_per_w // s_g

    @functools.partial(
        pl.kernel,
        out_type=(
            jax.ShapeDtypeStruct((b, d), dt),
            jax.ShapeDtypeStruct((b, d), dt),
        ),
        mesh=_sc_mesh(),
        scratch_types=[
            pltpu.VMEM((2, s_g), jnp.int32),
            pltpu.VMEM((s_g, d), dt),
            pltpu.VMEM((s_g, d), dt),
            pltpu.SemaphoreType.DMA,
            pltpu.SemaphoreType.DMA,
            pltpu.SemaphoreType.DMA,
            pltpu.SemaphoreType.DMA,
            pltpu.SemaphoreType.DMA,
            pltpu.SemaphoreType.DMA,
        ],
        compiler_params=pltpu.CompilerParams(use_tc_tiling_on_sc=False),
    )
    def k(table_hbm, ia_hbm, ib_hbm, oa_hbm, ob_hbm, idx_v, rows0, rows1,
          si0, si1, sg0, sg1, so0, so1):
        wid = lax.axis_index("s") * NC + lax.axis_index("c")
        base = wid * b_per_w
        rows = (rows0, rows1)
        si = (si0, si1)
        sg = (sg0, sg1)
        so = (so0, so1)

        chunks = []
        for idx_hbm, out_hbm in ((ia_hbm, oa_hbm), (ib_hbm, ob_hbm)):
            for j in range(n_ch):
                chunks.append((idx_hbm, out_hbm, base + j * s_g))

        n = len(chunks)
        gathers = [None] * n
        outs = [None] * n
        for j, (idx_hbm, out_hbm, off) in enumerate(chunks):
            bb = j & 1
            if j >= 2:
                outs[j - 2].wait()
            pltpu.async_copy(
                idx_hbm.at[pl.ds(off, s_g)], idx_v.at[bb], si[bb]
            ).wait()
            gathers[j] = pltpu.async_copy(
                table_hbm.at[idx_v.at[bb]], rows[bb], sg[bb]
            )
            if j >= 1:
                bp = (j - 1) & 1
                gathers[j - 1].wait()
                _, o_hbm, o_off = chunks[j - 1]
                outs[j - 1] = pltpu.async_copy(
                    rows[bp], o_hbm.at[pl.ds(o_off, s_g)], so[bp]
                )
        gathers[n - 1].wait()
        _, o_hbm, o_off = chunks[n - 1]
        outs[n - 1] = pltpu.async_copy(
            rows[(n - 1) & 1], o_hbm.at[pl.ds(o_off, s_g)], so[(n - 1) & 1]
        )
        outs[n - 2].wait()
        outs[n - 1].wait()

    return k(table, idx_a, idx_b)


def _sc_segsum(payload, idx, n_seg):
    e_tot = payload.shape[0]
    fc = 64 if n_seg * 64 * 4 <= 6 * 2**20 else 8
    n_fc = 128 // (NC * fc)
    e_pt = e_tot // NS
    s_s = 1000 if e_pt <= 10000 else 2000
    n_sub = e_pt // s_s
    rows = n_seg // NS
    idx3 = idx.reshape(NS, n_sub, s_s)
    zeros = jnp.zeros((n_seg, fc), F32)

    @functools.partial(
        pl.kernel,
        out_type=jax.ShapeDtypeStruct((n_seg, 128), F32),
        mesh=_sc_mesh(),
        scratch_types=[
            pltpu.VMEM((n_sub, s_s), jnp.int32),
            pltpu.VMEM((s_s, fc), F32),
            pltpu.VMEM_SHARED((n_seg, fc), F32),
        ],
        compiler_params=pltpu.CompilerParams(use_tc_tiling_on_sc=False),
    )
    def k(p_hbm, idx_hbm, z_hbm, out_hbm, idx_v, pbuf, shared):
        c = lax.axis_index("c")
        s = lax.axis_index("s")
        pltpu.sync_copy(idx_hbm.at[s], idx_v)
        for kk in range(n_fc):
            col0 = (kk * NC + c) * fc
            pltpu.sync_copy(
                z_hbm.at[pl.ds(s * rows, rows)], shared.at[pl.ds(s * rows, rows)]
            )
            plsc.subcore_barrier()

            def body(j, carry):
                pltpu.sync_copy(
                    p_hbm.at[pl.ds(s * e_pt + j * s_s, s_s), pl.ds(col0, fc)],
                    pbuf,
                )
                pltpu.sync_copy(pbuf, shared.at[idx_v.at[j]], add=True)
                return carry

            lax.fori_loop(0, n_sub, body, 0)
            plsc.subcore_barrier()
            pltpu.sync_copy(
                shared.at[pl.ds(s * rows, rows)],
                out_hbm.at[pl.ds(s * rows, rows), pl.ds(col0, fc)],
            )
            plsc.subcore_barrier()

    return k(payload, idx3, zeros)


def _tc_atom_embed(ids, emb):
    n = ids.shape[0]
    bm = 1000

    def body(ids_ref, emb_ref, o_ref, o16_ref):
        iot = lax.broadcasted_iota(jnp.int32, (bm, 108), 1)
        oh = (ids_ref[...] == iot).astype(F32)
        res = jnp.dot(oh, emb_ref[...], preferred_element_type=F32)
        o_ref[...] = res
        o16_ref[...] = res.astype(jnp.bfloat16)

    return pl.pallas_call(
        body,
        grid=(n // bm,),
        in_specs=[
            pl.BlockSpec((bm, 1), lambda i: (i, 0)),
            pl.BlockSpec((108, H), lambda i: (0, 0)),
        ],
        out_specs=[
            pl.BlockSpec((bm, H), lambda i: (i, 0)),
            pl.BlockSpec((bm, H), lambda i: (i, 0)),
        ],
        out_shape=[
            jax.ShapeDtypeStruct((n, H), F32),
            jax.ShapeDtypeStruct((n, H), jnp.bfloat16),
        ],
    )(ids.reshape(n, 1), emb)


def _rbf_block(v, vmin, vmax, bins):
    step = (vmax - vmin) / float(bins - 1)
    iot = lax.broadcasted_iota(jnp.int32, (1, bins), 1).astype(F32)
    centers = iot * step + vmin
    gamma = float(bins - 1) / (vmax - vmin)
    return jnp.exp(-((gamma * (v - centers)) ** 2))


def _mlp_block(f, w1, b1, g1, be1, w2, b2, g2, be2):
    h1 = _silu(_ln(jnp.dot(f, w1, preferred_element_type=F32) + b1, g1, be1))
    return _silu(_ln(jnp.dot(h1, w2, preferred_element_type=F32) + b2, g2, be2))


def _tc_edge_embed(r16, e_w1, e_b1, e_ln1, e_w2, e_b2, e_ln2):
    e = r16.shape[0]
    bm = 2000

    def body(r_ref, w1, b1, g1, be1, w2, b2, g2, be2, y_ref, q_ref, c_ref):
        rv = r_ref[...]
        bl2 = jnp.sum(rv * rv, axis=1, keepdims=True)
        bl = jnp.sqrt(bl2)
        y_ref[...] = _mlp_block(
            _rbf_block(bl, 0.0, 8.0, 80),
            w1[...], b1[...], g1[...], be1[...], w2[...], b2[...], g2[...], be2[...],
        )
        ci = lax.broadcasted_iota(jnp.int32, (bm, 16), 1)
        q_ref[...] = jnp.where(ci == 3, bl, rv)
        r_in, r_out = 7.5, 8.0
        sw = ((r_out**2 - bl2) ** 2 * (r_out**2 + 2.0 * bl2 - 3.0 * r_in**2)) / (
            r_out**2 - r_in**2
        ) ** 3
        c_ref[...] = jnp.where(bl < r_in, 1.0, jnp.where(bl > r_out, 0.0, sw))

    vec = lambda: pl.BlockSpec((1, H), lambda i: (0, 0))
    return pl.pallas_call(
        body,
        grid=(e // bm,),
        in_specs=[
            pl.BlockSpec((bm, 16), lambda i: (i, 0)),
            pl.BlockSpec((80, H), lambda i: (0, 0)),
            vec(), vec(), vec(),
            pl.BlockSpec((H, H), lambda i: (0, 0)),
            vec(), vec(), vec(),
        ],
        out_specs=[
            pl.BlockSpec((bm, H), lambda i: (i, 0)),
            pl.BlockSpec((bm, 16), lambda i: (i, 0)),
            pl.BlockSpec((bm, 1), lambda i: (i, 0)),
        ],
        out_shape=[
            jax.ShapeDtypeStruct((e, H), F32),
            jax.ShapeDtypeStruct((e, 16), F32),
            jax.ShapeDtypeStruct((e, 1), F32),
        ],
    )(
        r16,
        e_w1, e_b1.reshape(1, H), e_ln1[0].reshape(1, H), e_ln1[1].reshape(1, H),
        e_w2, e_b2.reshape(1, H), e_ln2[0].reshape(1, H), e_ln2[1].reshape(1, H),
    )


def _tc_angle_embed(qs, qd, a_w1, a_b1, a_ln1, a_w2, a_b2, a_ln2):
    lg = qs.shape[0]
    bm = 2000

    def body(qs_ref, qd_ref, w1, b1, g1, be1, w2, b2, g2, be2, z_ref):
        a = qs_ref[...]
        b = qd_ref[...]
        ci = lax.broadcasted_iota(jnp.int32, (bm, 16), 1)
        prod = jnp.where(ci < 3, a * b, 0.0)
        d3 = jnp.sum(prod, axis=1, keepdims=True)
        l1 = a[:, 3:4]
        l2 = b[:, 3:4]
        cos = -d3 / (l1 * l2 + 1e-8)
        z_ref[...] = _mlp_block(
            _rbf_block(cos, -1.0, 1.0, 40),
            w1[...], b1[...], g1[...], be1[...], w2[...], b2[...], g2[...], be2[...],
        )

    vec = lambda: pl.BlockSpec((1, H), lambda i: (0, 0))
    return pl.pallas_call(
        body,
        grid=(lg // bm,),
        in_specs=[
            pl.BlockSpec((bm, 16), lambda i: (i, 0)),
            pl.BlockSpec((bm, 16), lambda i: (i, 0)),
            pl.BlockSpec((40, H), lambda i: (0, 0)),
            vec(), vec(), vec(),
            pl.BlockSpec((H, H), lambda i: (0, 0)),
            vec(), vec(), vec(),
        ],
        out_specs=pl.BlockSpec((bm, H), lambda i: (i, 0)),
        out_shape=jax.ShapeDtypeStruct((lg, H), F32),
    )(
        qs, qd,
        a_w1, a_b1.reshape(1, H), a_ln1[0].reshape(1, H), a_ln1[1].reshape(1, H),
        a_w2, a_b2.reshape(1, H), a_ln2[0].reshape(1, H), a_ln2[1].reshape(1, H),
    )


def _tc_edge_conv(ga, gb, y, cut, w, b, ln, has_cutoff, want_y16):
    e = ga.shape[0]
    bm = 2000
    bsum = (b[0] + b[1] + b[2]).reshape(1, H)
    b3 = b[3].reshape(1, H)

    def body(*refs):
        y16_ref = None
        if want_y16:
            refs, y16_ref = refs[:-1], refs[-1]
        if has_cutoff:
            (ga_r, gb_r, y_r, c_r, w0, w1, w2, w3, bs, b3r, g_r, be_r,
             p_ref, yn_ref) = refs
        else:
            (ga_r, gb_r, y_r, w0, w1, w2, w3, bs, b3r, g_r, be_r,
             p_ref, yn_ref) = refs
        av = ga_r[...].astype(F32)
        ev = (
            jnp.dot(av, w0[...], preferred_element_type=F32)
            + jnp.dot(gb_r[...].astype(F32), w1[...], preferred_element_type=F32)
            + jnp.dot(y_r[...], w2[...], preferred_element_type=F32)
            + bs[...]
        )
        sig = jax.nn.sigmoid(ev)
        if has_cutoff:
            sig = sig * c_r[...]
        bh = jnp.dot(av, w3[...], preferred_element_type=F32) + b3r[...]
        p_ref[:, :H] = sig * bh
        p_ref[:, H:] = sig
        ynv = y_r[...] + _silu(_ln(ev, g_r[...], be_r[...]))
        yn_ref[...] = ynv
        if want_y16:
            y16_ref[...] = ynv.astype(jnp.bfloat16)

    blk = lambda: pl.BlockSpec((bm, H), lambda i: (i, 0))
    mat = lambda: pl.BlockSpec((H, H), lambda i: (0, 0))
    vec = lambda: pl.BlockSpec((1, H), lambda i: (0, 0))
    in_specs = [blk(), blk(), blk()]
    args = [ga, gb, y]
    if has_cutoff:
        in_specs.append(pl.BlockSpec((bm, 1), lambda i: (i, 0)))
        args.append(cut)
    in_specs += [mat(), mat(), mat(), mat(), vec(), vec(), vec(), vec()]
    args += [w[0], w[1], w[2], w[3], bsum, b3,
             ln[2].reshape(1, H), ln[3].reshape(1, H)]
    out_specs = [
        pl.BlockSpec((bm, 2 * H), lambda i: (i, 0)),
        pl.BlockSpec((bm, H), lambda i: (i, 0)),
    ]
    out_shape = [
        jax.ShapeDtypeStruct((e, 2 * H), F32),
        jax.ShapeDtypeStruct((e, H), F32),
    ]
    if want_y16:
        out_specs.append(pl.BlockSpec((bm, H), lambda i: (i, 0)))
        out_shape.append(jax.ShapeDtypeStruct((e, H), jnp.bfloat16))
    return pl.pallas_call(
        body,
        grid=(e // bm,),
        in_specs=in_specs,
        out_specs=out_specs,
        out_shape=out_shape,
    )(*args)


def _tc_update(s, x, w4, b4, g, be):
    n = x.shape[0]
    bm = 1000 if n <= 10000 else 2000

    def body(s_ref, x_ref, w_r, b_r, g_r, be_r, o_ref, o16_ref):
        sv = s_ref[...]
        h = sv[:, :H] / (sv[:, H:] + 1e-6)
        xv = x_ref[...]
        res = xv + _silu(_ln(pre := jnp.dot(xv, w_r[...], preferred_element_type=F32) + b_r[...] + h, g_r[...], be_r[...]))
        o_ref[...] = res
        o16_ref[...] = res.astype(jnp.bfloat16)

    return pl.pallas_call(
        body,
        grid=(n // bm,),
        in_specs=[
            pl.BlockSpec((bm, 2 * H), lambda i: (i, 0)),
            pl.BlockSpec((bm, H), lambda i: (i, 0)),
            pl.BlockSpec((H, H), lambda i: (0, 0)),
            pl.BlockSpec((1, H), lambda i: (0, 0)),
            pl.BlockSpec((1, H), lambda i: (0, 0)),
            pl.BlockSpec((1, H), lambda i: (0, 0)),
        ],
        out_specs=[
            pl.BlockSpec((bm, H), lambda i: (i, 0)),
            pl.BlockSpec((bm, H), lambda i: (i, 0)),
        ],
        out_shape=[
            jax.ShapeDtypeStruct((n, H), F32),
            jax.ShapeDtypeStruct((n, H), jnp.bfloat16),
        ],
    )(s, x, w4, b4.reshape(1, H), g.reshape(1, H), be.reshape(1, H))


def _tc_final(x, fc_w, fc_b):
    n = x.shape[0]

    def body(x_ref, w_r, b_r, o_ref):
        m = jnp.mean(x_ref[...], axis=0, keepdims=True)
        o_ref[...] = jnp.dot(m, w_r[...], preferred_element_type=F32) + b_r[...]

    out = pl.pallas_call(
        body,
        grid=(1,),
        in_specs=[
            pl.BlockSpec((n, H), lambda i: (0, 0)),
            pl.BlockSpec((H, 1), lambda i: (0, 0)),
            pl.BlockSpec((1, 1), lambda i: (0, 0)),
        ],
        out_specs=pl.BlockSpec((1, 1), lambda i: (0, 0)),
        out_shape=jax.ShapeDtypeStruct((1, 1), F32),
    )(x, fc_w, fc_b.reshape(1, 1))
    return out[0, 0]


def _conv(x, x16, y, src, dst, n_seg, w, b, ln, cut, want_y16):
    ga, gb = _sc_gather2(x16, src, dst)
    outs = _tc_edge_conv(ga, gb, y, cut, w, b, ln, cut is not None, want_y16)
    p, y_new = outs[0], outs[1]
    y16 = outs[2] if want_y16 else None
    s = _sc_segsum(p, dst, n_seg)
    x_new, x16_new = _tc_update(s, x, w[4], b[4], ln[0], ln[1])
    return x_new, x16_new, y_new, y16


def kernel(r, atom_emb, e_W1, e_b1, e_ln1, e_W2, e_b2, e_ln2, a_W1, a_b1,
           a_ln1, a_W2, a_b2, a_ln2, conv_W, conv_b, conv_ln, fc_W, fc_b,
           atomic_number, edge_index, lg_edge_index):
    n_nodes = atomic_number.shape[0]
    n_edges = r.shape[0]
    src, dst = edge_index[0], edge_index[1]
    lsrc, ldst = lg_edge_index[0], lg_edge_index[1]

    r16 = jnp.pad(r.astype(F32), ((0, 0), (0, 13)))
    x, x16 = _tc_atom_embed(atomic_number.astype(jnp.int32), atom_emb)
    y, q, cut = _tc_edge_embed(r16, e_W1, e_b1, e_ln1, e_W2, e_b2, e_ln2)
    qs, qd = _sc_gather2(q, lsrc, ldst)
    z = _tc_angle_embed(qs, qd, a_W1, a_b1, a_ln1, a_W2, a_b2, a_ln2)

    li = 0
    for _ in range(2):
        x, x16, m, m16 = _conv(x, x16, y, src, dst, n_nodes, conv_W[li],
                               conv_b[li], conv_ln[li], cut, True)
        li += 1
        y, _, z, _ = _conv(m, m16, z, lsrc, ldst, n_edges, conv_W[li],
                           conv_b[li], conv_ln[li], None, False)
        li += 1
    for _ in range(2):
        x, x16, y, _ = _conv(x, x16, y, src, dst, n_nodes, conv_W[li],
                             conv_b[li], conv_ln[li], cut, False)
        li += 1
    return _tc_final(x, fc_W, fc_b)


# speedup vs baseline: 1.0762x; 1.0762x over previous
import functools

import jax
import jax.numpy as jnp
from jax import lax
from jax.experimental import pallas as pl
from jax.experimental.pallas import tpu as pltpu
from jax.experimental.pallas import tpu_sc as plsc

NC = 2
NS = 16
NW = NC * NS
H = 64
F32 = jnp.float32


def _silu(v):
    return v * jax.nn.sigmoid(v)


def _ln(v, g, b, eps=1e-5):
    mu = jnp.mean(v, axis=-1, keepdims=True)
    var = jnp.mean((v - mu) ** 2, axis=-1, keepdims=True)
    return (v - mu) / jnp.sqrt(var + eps) * g + b


def _sc_mesh():
    return plsc.VectorSubcoreMesh(
        core_axis_name="c", subcore_axis_name="s", num_cores=NC, num_subcores=NS
    )


def _sc_gather2(table, idx_a, idx_b):
    v, d = table.shape
    dt = table.dtype
    b = idx_a.shape[0]
    b_per_w = b // NW
    ns = 5
    s_g = 200
    grp = ns * s_g
    n_grp = b_per_w // grp

    @functools.partial(
        pl.kernel,
        out_type=(
            jax.ShapeDtypeStruct((b, d), dt),
            jax.ShapeDtypeStruct((b, d), dt),
        ),
        mesh=_sc_mesh(),
        scratch_types=[
            pltpu.VMEM((grp,), jnp.int32),
            pltpu.VMEM((grp, d), dt),
            pltpu.SemaphoreType.DMA,
            pltpu.SemaphoreType.DMA,
            pltpu.SemaphoreType.DMA,
            pltpu.SemaphoreType.DMA,
            pltpu.SemaphoreType.DMA,
            pltpu.SemaphoreType.DMA,
        ],
        compiler_params=pltpu.CompilerParams(use_tc_tiling_on_sc=False),
    )
    def k(table_hbm, ia_hbm, ib_hbm, oa_hbm, ob_hbm, idx_v, rows_v,
          sg0, sg1, sg2, sg3, sg4, so):
        wid = lax.axis_index("s") * NC + lax.axis_index("c")
        base = wid * b_per_w
        sg = (sg0, sg1, sg2, sg3, sg4)

        for idx_hbm, out_hbm in ((ia_hbm, oa_hbm), (ib_hbm, ob_hbm)):
            def group(g, carry):
                goff = base + g * grp
                pltpu.sync_copy(idx_hbm.at[pl.ds(goff, grp)], idx_v)
                cps = []
                for t in range(ns):
                    cps.append(pltpu.async_copy(
                        table_hbm.at[idx_v.at[pl.ds(t * s_g, s_g)]],
                        rows_v.at[pl.ds(t * s_g, s_g)],
                        sg[t],
                    ))
                for t in range(ns):
                    cps[t].wait()
                pltpu.async_copy(
                    rows_v, out_hbm.at[pl.ds(goff, grp)], so
                ).wait()
                return carry

            lax.fori_loop(0, n_grp, group, 0)

    return k(table, idx_a, idx_b)


def _sc_segsum(payload, idx, n_seg):
    e_tot = payload.shape[0]
    fc = 64 if n_seg * 64 * 4 <= 6 * 2**20 else 8
    n_fc = 128 // (NC * fc)
    e_pt = e_tot // NS
    s_s = 1000 if e_pt <= 10000 else 2000
    n_sub = e_pt // s_s
    rows = n_seg // NS
    idx3 = idx.reshape(NS, n_sub, s_s)
    zeros = jnp.zeros((n_seg, fc), F32)

    @functools.partial(
        pl.kernel,
        out_type=jax.ShapeDtypeStruct((n_seg, 128), F32),
        mesh=_sc_mesh(),
        scratch_types=[
            pltpu.VMEM((n_sub, s_s), jnp.int32),
            pltpu.VMEM((s_s, fc), F32),
            pltpu.VMEM_SHARED((n_seg, fc), F32),
        ],
        compiler_params=pltpu.CompilerParams(use_tc_tiling_on_sc=False),
    )
    def k(p_hbm, idx_hbm, z_hbm, out_hbm, idx_v, pbuf, shared):
        c = lax.axis_index("c")
        s = lax.axis_index("s")
        pltpu.sync_copy(idx_hbm.at[s], idx_v)
        for kk in range(n_fc):
            col0 = (kk * NC + c) * fc
            pltpu.sync_copy(
                z_hbm.at[pl.ds(s * rows, rows)], shared.at[pl.ds(s * rows, rows)]
            )
            plsc.subcore_barrier()

            def body(j, carry):
                pltpu.sync_copy(
                    p_hbm.at[pl.ds(s * e_pt + j * s_s, s_s), pl.ds(col0, fc)],
                    pbuf,
                )
                pltpu.sync_copy(pbuf, shared.at[idx_v.at[j]], add=True)
                return carry

            lax.fori_loop(0, n_sub, body, 0)
            plsc.subcore_barrier()
            pltpu.sync_copy(
                shared.at[pl.ds(s * rows, rows)],
                out_hbm.at[pl.ds(s * rows, rows), pl.ds(col0, fc)],
            )
            plsc.subcore_barrier()

    return k(payload, idx3, zeros)


def _tc_atom_embed(ids, emb):
    n = ids.shape[0]
    bm = 1000

    def body(ids_ref, emb_ref, o_ref):
        iot = lax.broadcasted_iota(jnp.int32, (bm, 108), 1)
        oh = (ids_ref[...] == iot).astype(F32)
        o_ref[...] = jnp.dot(oh, emb_ref[...], preferred_element_type=F32)

    return pl.pallas_call(
        body,
        grid=(n // bm,),
        in_specs=[
            pl.BlockSpec((bm, 1), lambda i: (i, 0)),
            pl.BlockSpec((108, H), lambda i: (0, 0)),
        ],
        out_specs=pl.BlockSpec((bm, H), lambda i: (i, 0)),
        out_shape=jax.ShapeDtypeStruct((n, H), F32),
    )(ids.reshape(n, 1), emb)


def _rbf_block(v, vmin, vmax, bins):
    step = (vmax - vmin) / float(bins - 1)
    iot = lax.broadcasted_iota(jnp.int32, (1, bins), 1).astype(F32)
    centers = iot * step + vmin
    gamma = float(bins - 1) / (vmax - vmin)
    return jnp.exp(-((gamma * (v - centers)) ** 2))


def _mlp_block(f, w1, b1, g1, be1, w2, b2, g2, be2):
    h1 = _silu(_ln(jnp.dot(f, w1, preferred_element_type=F32) + b1, g1, be1))
    return _silu(_ln(jnp.dot(h1, w2, preferred_element_type=F32) + b2, g2, be2))


def _tc_edge_embed(r16, e_w1, e_b1, e_ln1, e_w2, e_b2, e_ln2):
    e = r16.shape[0]
    bm = 2000

    def body(r_ref, w1, b1, g1, be1, w2, b2, g2, be2, y_ref, q_ref, c_ref):
        rv = r_ref[...]
        bl2 = jnp.sum(rv * rv, axis=1, keepdims=True)
        bl = jnp.sqrt(bl2)
        y_ref[...] = _mlp_block(
            _rbf_block(bl, 0.0, 8.0, 80),
            w1[...], b1[...], g1[...], be1[...], w2[...], b2[...], g2[...], be2[...],
        )
        ci = lax.broadcasted_iota(jnp.int32, (bm, 16), 1)
        q_ref[...] = jnp.where(ci == 3, bl, rv)
        r_in, r_out = 7.5, 8.0
        sw = ((r_out**2 - bl2) ** 2 * (r_out**2 + 2.0 * bl2 - 3.0 * r_in**2)) / (
            r_out**2 - r_in**2
        ) ** 3
        c_ref[...] = jnp.where(bl < r_in, 1.0, jnp.where(bl > r_out, 0.0, sw))

    vec = lambda: pl.BlockSpec((1, H), lambda i: (0, 0))
    return pl.pallas_call(
        body,
        grid=(e // bm,),
        in_specs=[
            pl.BlockSpec((bm, 16), lambda i: (i, 0)),
            pl.BlockSpec((80, H), lambda i: (0, 0)),
            vec(), vec(), vec(),
            pl.BlockSpec((H, H), lambda i: (0, 0)),
            vec(), vec(), vec(),
        ],
        out_specs=[
            pl.BlockSpec((bm, H), lambda i: (i, 0)),
            pl.BlockSpec((bm, 16), lambda i: (i, 0)),
            pl.BlockSpec((bm, 1), lambda i: (i, 0)),
        ],
        out_shape=[
            jax.ShapeDtypeStruct((e, H), F32),
            jax.ShapeDtypeStruct((e, 16), F32),
            jax.ShapeDtypeStruct((e, 1), F32),
        ],
    )(
        r16,
        e_w1, e_b1.reshape(1, H), e_ln1[0].reshape(1, H), e_ln1[1].reshape(1, H),
        e_w2, e_b2.reshape(1, H), e_ln2[0].reshape(1, H), e_ln2[1].reshape(1, H),
    )


def _tc_angle_embed(qs, qd, a_w1, a_b1, a_ln1, a_w2, a_b2, a_ln2):
    lg = qs.shape[0]
    bm = 2000

    def body(qs_ref, qd_ref, w1, b1, g1, be1, w2, b2, g2, be2, z_ref):
        a = qs_ref[...]
        b = qd_ref[...]
        ci = lax.broadcasted_iota(jnp.int32, (bm, 16), 1)
        prod = jnp.where(ci < 3, a * b, 0.0)
        d3 = jnp.sum(prod, axis=1, keepdims=True)
        l1 = a[:, 3:4]
        l2 = b[:, 3:4]
        cos = -d3 / (l1 * l2 + 1e-8)
        z_ref[...] = _mlp_block(
            _rbf_block(cos, -1.0, 1.0, 40),
            w1[...], b1[...], g1[...], be1[...], w2[...], b2[...], g2[...], be2[...],
        )

    vec = lambda: pl.BlockSpec((1, H), lambda i: (0, 0))
    return pl.pallas_call(
        body,
        grid=(lg // bm,),
        in_specs=[
            pl.BlockSpec((bm, 16), lambda i: (i, 0)),
            pl.BlockSpec((bm, 16), lambda i: (i, 0)),
            pl.BlockSpec((40, H), lambda i: (0, 0)),
            vec(), vec(), vec(),
            pl.BlockSpec((H, H), lambda i: (0, 0)),
            vec(), vec(), vec(),
        ],
        out_specs=pl.BlockSpec((bm, H), lambda i: (i, 0)),
        out_shape=jax.ShapeDtypeStruct((lg, H), F32),
    )(
        qs, qd,
        a_w1, a_b1.reshape(1, H), a_ln1[0].reshape(1, H), a_ln1[1].reshape(1, H),
        a_w2, a_b2.reshape(1, H), a_ln2[0].reshape(1, H), a_ln2[1].reshape(1, H),
    )


def _tc_edge_conv(ga, gb, y, cut, w, b, ln, has_cutoff):
    e = ga.shape[0]
    bm = 2000
    bsum = (b[0] + b[1] + b[2]).reshape(1, H)
    b3 = b[3].reshape(1, H)

    def body(*refs):
        if has_cutoff:
            (ga_r, gb_r, y_r, c_r, w0, w1, w2, w3, bs, b3r, g_r, be_r,
             p_ref, yn_ref) = refs
        else:
            (ga_r, gb_r, y_r, w0, w1, w2, w3, bs, b3r, g_r, be_r,
             p_ref, yn_ref) = refs
        av = ga_r[...]
        ev = (
            jnp.dot(av, w0[...], preferred_element_type=F32)
            + jnp.dot(gb_r[...], w1[...], preferred_element_type=F32)
            + jnp.dot(y_r[...], w2[...], preferred_element_type=F32)
            + bs[...]
        )
        sig = jax.nn.sigmoid(ev)
        if has_cutoff:
            sig = sig * c_r[...]
        bh = jnp.dot(av, w3[...], preferred_element_type=F32) + b3r[...]
        p_ref[:, :H] = sig * bh
        p_ref[:, H:] = sig
        yn_ref[...] = y_r[...] + _silu(_ln(ev, g_r[...], be_r[...]))

    blk = lambda: pl.BlockSpec((bm, H), lambda i: (i, 0))
    mat = lambda: pl.BlockSpec((H, H), lambda i: (0, 0))
    vec = lambda: pl.BlockSpec((1, H), lambda i: (0, 0))
    in_specs = [blk(), blk(), blk()]
    args = [ga, gb, y]
    if has_cutoff:
        in_specs.append(pl.BlockSpec((bm, 1), lambda i: (i, 0)))
        args.append(cut)
    in_specs += [mat(), mat(), mat(), mat(), vec(), vec(), vec(), vec()]
    args += [w[0], w[1], w[2], w[3], bsum, b3,
             ln[2].reshape(1, H), ln[3].reshape(1, H)]
    return pl.pallas_call(
        body,
        grid=(e // bm,),
        in_specs=in_specs,
        out_specs=[
            pl.BlockSpec((bm, 2 * H), lambda i: (i, 0)),
            pl.BlockSpec((bm, H), lambda i: (i, 0)),
        ],
        out_shape=[
            jax.ShapeDtypeStruct((e, 2 * H), F32),
            jax.ShapeDtypeStruct((e, H), F32),
        ],
    )(*args)


def _tc_update(s, x, w4, b4, g, be):
    n = x.shape[0]
    bm = 1000 if n <= 10000 else 2000

    def body(s_ref, x_ref, w_r, b_r, g_r, be_r, o_ref):
        sv = s_ref[...]
        h = sv[:, :H] / (sv[:, H:] + 1e-6)
        xv = x_ref[...]
        pre = jnp.dot(xv, w_r[...], preferred_element_type=F32) + b_r[...] + h
        o_ref[...] = xv + _silu(_ln(pre, g_r[...], be_r[...]))

    return pl.pallas_call(
        body,
        grid=(n // bm,),
        in_specs=[
            pl.BlockSpec((bm, 2 * H), lambda i: (i, 0)),
            pl.BlockSpec((bm, H), lambda i: (i, 0)),
            pl.BlockSpec((H, H), lambda i: (0, 0)),
            pl.BlockSpec((1, H), lambda i: (0, 0)),
            pl.BlockSpec((1, H), lambda i: (0, 0)),
            pl.BlockSpec((1, H), lambda i: (0, 0)),
        ],
        out_specs=pl.BlockSpec((bm, H), lambda i: (i, 0)),
        out_shape=jax.ShapeDtypeStruct((n, H), F32),
    )(s, x, w4, b4.reshape(1, H), g.reshape(1, H), be.reshape(1, H))


def _tc_final(x, fc_w, fc_b):
    n = x.shape[0]

    def body(x_ref, w_r, b_r, o_ref):
        m = jnp.mean(x_ref[...], axis=0, keepdims=True)
        o_ref[...] = jnp.dot(m, w_r[...], preferred_element_type=F32) + b_r[...]

    out = pl.pallas_call(
        body,
        grid=(1,),
        in_specs=[
            pl.BlockSpec((n, H), lambda i: (0, 0)),
            pl.BlockSpec((H, 1), lambda i: (0, 0)),
            pl.BlockSpec((1, 1), lambda i: (0, 0)),
        ],
        out_specs=pl.BlockSpec((1, 1), lambda i: (0, 0)),
        out_shape=jax.ShapeDtypeStruct((1, 1), F32),
    )(x, fc_w, fc_b.reshape(1, 1))
    return out[0, 0]


def _conv(x, y, src, dst, n_seg, w, b, ln, cut):
    ga, gb = _sc_gather2(x, src, dst)
    p, y_new = _tc_edge_conv(ga, gb, y, cut, w, b, ln, cut is not None)
    s = _sc_segsum(p, dst, n_seg)
    x_new = _tc_update(s, x, w[4], b[4], ln[0], ln[1])
    return x_new, y_new


def kernel(r, atom_emb, e_W1, e_b1, e_ln1, e_W2, e_b2, e_ln2, a_W1, a_b1,
           a_ln1, a_W2, a_b2, a_ln2, conv_W, conv_b, conv_ln, fc_W, fc_b,
           atomic_number, edge_index, lg_edge_index):
    n_nodes = atomic_number.shape[0]
    n_edges = r.shape[0]
    src, dst = edge_index[0], edge_index[1]
    lsrc, ldst = lg_edge_index[0], lg_edge_index[1]

    r16 = jnp.pad(r.astype(F32), ((0, 0), (0, 13)))
    x = _tc_atom_embed(atomic_number.astype(jnp.int32), atom_emb)
    y, q, cut = _tc_edge_embed(r16, e_W1, e_b1, e_ln1, e_W2, e_b2, e_ln2)
    qs, qd = _sc_gather2(q, lsrc, ldst)
    z = _tc_angle_embed(qs, qd, a_W1, a_b1, a_ln1, a_W2, a_b2, a_ln2)

    li = 0
    for _ in range(2):
        x, m = _conv(x, y, src, dst, n_nodes, conv_W[li], conv_b[li],
                     conv_ln[li], cut)
        li += 1
        y, z = _conv(m, z, lsrc, ldst, n_edges, conv_W[li], conv_b[li],
                     conv_ln[li], None)
        li += 1
    for _ in range(2):
        x, y = _conv(x, y, src, dst, n_nodes, conv_W[li], conv_b[li],
                     conv_ln[li], cut)
        li += 1
    return _tc_final(x, fc_W, fc_b)


# speedup vs baseline: 1.1193x; 1.0400x over previous
import functools

import jax
import jax.numpy as jnp
from jax import lax
from jax.experimental import pallas as pl
from jax.experimental.pallas import tpu as pltpu
from jax.experimental.pallas import tpu_sc as plsc

NC = 2
NS = 16
NW = NC * NS
H = 64
F32 = jnp.float32


def _silu(v):
    return v * jax.nn.sigmoid(v)


def _ln(v, g, b, eps=1e-5):
    mu = jnp.mean(v, axis=-1, keepdims=True)
    var = jnp.mean((v - mu) ** 2, axis=-1, keepdims=True)
    return (v - mu) / jnp.sqrt(var + eps) * g + b


def _sc_mesh():
    return plsc.VectorSubcoreMesh(
        core_axis_name="c", subcore_axis_name="s", num_cores=NC, num_subcores=NS
    )


def _sc_gather2(table, idx_a, idx_b):
    v, d = table.shape
    dt = table.dtype
    b = idx_a.shape[0]
    b_per_w = b // NW
    ns = 5
    s_g = 200
    grp = ns * s_g
    n_grp = b_per_w // grp

    @functools.partial(
        pl.kernel,
        out_type=(
            jax.ShapeDtypeStruct((b, d), dt),
            jax.ShapeDtypeStruct((b, d), dt),
        ),
        mesh=_sc_mesh(),
        scratch_types=[
            pltpu.VMEM((grp,), jnp.int32),
            pltpu.VMEM((grp, d), dt),
            pltpu.SemaphoreType.DMA,
            pltpu.SemaphoreType.DMA,
            pltpu.SemaphoreType.DMA,
            pltpu.SemaphoreType.DMA,
            pltpu.SemaphoreType.DMA,
            pltpu.SemaphoreType.DMA,
        ],
        compiler_params=pltpu.CompilerParams(use_tc_tiling_on_sc=False),
    )
    def k(table_hbm, ia_hbm, ib_hbm, oa_hbm, ob_hbm, idx_v, rows_v,
          sg0, sg1, sg2, sg3, sg4, so):
        wid = lax.axis_index("s") * NC + lax.axis_index("c")
        base = wid * b_per_w
        sg = (sg0, sg1, sg2, sg3, sg4)

        for idx_hbm, out_hbm in ((ia_hbm, oa_hbm), (ib_hbm, ob_hbm)):
            def group(g, carry):
                goff = base + g * grp
                pltpu.sync_copy(idx_hbm.at[pl.ds(goff, grp)], idx_v)
                cps = []
                for t in range(ns):
                    cps.append(pltpu.async_copy(
                        table_hbm.at[idx_v.at[pl.ds(t * s_g, s_g)]],
                        rows_v.at[pl.ds(t * s_g, s_g)],
                        sg[t],
                    ))
                for t in range(ns):
                    cps[t].wait()
                pltpu.async_copy(
                    rows_v, out_hbm.at[pl.ds(goff, grp)], so
                ).wait()
                return carry

            lax.fori_loop(0, n_grp, group, 0)

    return k(table, idx_a, idx_b)


def _sc_segsum(payload, idx, n_seg):
    e_tot = payload.shape[0]
    fc = 64 if n_seg * 64 * 4 <= 6 * 2**20 else 8
    n_fc = 128 // (NC * fc)
    e_pt = e_tot // NS
    s_s = 200 if e_pt <= 10000 else 2000
    n_sub = e_pt // s_s
    rows = n_seg // NS
    idx3 = idx.reshape(NS, n_sub, s_s)
    zeros = jnp.zeros((n_seg, fc), F32)

    @functools.partial(
        pl.kernel,
        out_type=jax.ShapeDtypeStruct((n_seg, 128), F32),
        mesh=_sc_mesh(),
        scratch_types=[
            pltpu.VMEM((2, s_s), jnp.int32),
            pltpu.VMEM((s_s, fc), F32),
            pltpu.VMEM((s_s, fc), F32),
            pltpu.VMEM_SHARED((n_seg, fc), F32),
            pltpu.SemaphoreType.DMA,
            pltpu.SemaphoreType.DMA,
            pltpu.SemaphoreType.DMA,
            pltpu.SemaphoreType.DMA,
        ],
        compiler_params=pltpu.CompilerParams(use_tc_tiling_on_sc=False),
    )
    def k(p_hbm, idx_hbm, z_hbm, out_hbm, idx_v, pbuf0, pbuf1, shared,
          si0, si1, sp0, sp1):
        c = lax.axis_index("c")
        s = lax.axis_index("s")
        pbuf = (pbuf0, pbuf1)
        si = (si0, si1)
        sp = (sp0, sp1)

        for kk in range(n_fc):
            col0 = (kk * NC + c) * fc

            def start(j, bpar):
                pltpu.async_copy(idx_hbm.at[s, j], idx_v.at[bpar], si[bpar])
                pltpu.async_copy(
                    p_hbm.at[pl.ds(s * e_pt + j * s_s, s_s), pl.ds(col0, fc)],
                    pbuf[bpar],
                    sp[bpar],
                )

            def finish(j, bpar):
                pltpu.make_async_copy(
                    idx_hbm.at[s, j], idx_v.at[bpar], si[bpar]
                ).wait()
                pltpu.make_async_copy(
                    p_hbm.at[pl.ds(s * e_pt + j * s_s, s_s), pl.ds(col0, fc)],
                    pbuf[bpar],
                    sp[bpar],
                ).wait()
                pltpu.sync_copy(pbuf[bpar], shared.at[idx_v.at[bpar]], add=True)

            pltpu.sync_copy(
                z_hbm.at[pl.ds(s * rows, rows)], shared.at[pl.ds(s * rows, rows)]
            )
            plsc.subcore_barrier()

            start(0, 0)
            start(1, 1)

            def body(j2, carry):
                for bpar in (0, 1):
                    j = j2 * 2 + bpar
                    finish(j, bpar)

                    @pl.when(j + 2 < n_sub)
                    def _():
                        start(j + 2, bpar)

                return carry

            lax.fori_loop(0, n_sub // 2, body, 0)
            plsc.subcore_barrier()
            pltpu.sync_copy(
                shared.at[pl.ds(s * rows, rows)],
                out_hbm.at[pl.ds(s * rows, rows), pl.ds(col0, fc)],
            )
            plsc.subcore_barrier()

    return k(payload, idx3, zeros)


def _tc_atom_embed(ids, emb):
    n = ids.shape[0]
    bm = 1000

    def body(ids_ref, emb_ref, o_ref):
        iot = lax.broadcasted_iota(jnp.int32, (bm, 108), 1)
        oh = (ids_ref[...] == iot).astype(F32)
        o_ref[...] = jnp.dot(oh, emb_ref[...], preferred_element_type=F32)

    return pl.pallas_call(
        body,
        grid=(n // bm,),
        in_specs=[
            pl.BlockSpec((bm, 1), lambda i: (i, 0)),
            pl.BlockSpec((108, H), lambda i: (0, 0)),
        ],
        out_specs=pl.BlockSpec((bm, H), lambda i: (i, 0)),
        out_shape=jax.ShapeDtypeStruct((n, H), F32),
    )(ids.reshape(n, 1), emb)


def _rbf_block(v, vmin, vmax, bins):
    step = (vmax - vmin) / float(bins - 1)
    iot = lax.broadcasted_iota(jnp.int32, (1, bins), 1).astype(F32)
    centers = iot * step + vmin
    gamma = float(bins - 1) / (vmax - vmin)
    return jnp.exp(-((gamma * (v - centers)) ** 2))


def _mlp_block(f, w1, b1, g1, be1, w2, b2, g2, be2):
    h1 = _silu(_ln(jnp.dot(f, w1, preferred_element_type=F32) + b1, g1, be1))
    return _silu(_ln(jnp.dot(h1, w2, preferred_element_type=F32) + b2, g2, be2))


def _tc_edge_embed(r16, e_w1, e_b1, e_ln1, e_w2, e_b2, e_ln2):
    e = r16.shape[0]
    bm = 2000

    def body(r_ref, w1, b1, g1, be1, w2, b2, g2, be2, y_ref, q_ref, c_ref):
        rv = r_ref[...]
        bl2 = jnp.sum(rv * rv, axis=1, keepdims=True)
        bl = jnp.sqrt(bl2)
        y_ref[...] = _mlp_block(
            _rbf_block(bl, 0.0, 8.0, 80),
            w1[...], b1[...], g1[...], be1[...], w2[...], b2[...], g2[...], be2[...],
        )
        ci = lax.broadcasted_iota(jnp.int32, (bm, 16), 1)
        q_ref[...] = jnp.where(ci == 3, bl, rv)
        r_in, r_out = 7.5, 8.0
        sw = ((r_out**2 - bl2) ** 2 * (r_out**2 + 2.0 * bl2 - 3.0 * r_in**2)) / (
            r_out**2 - r_in**2
        ) ** 3
        c_ref[...] = jnp.where(bl < r_in, 1.0, jnp.where(bl > r_out, 0.0, sw))

    vec = lambda: pl.BlockSpec((1, H), lambda i: (0, 0))
    return pl.pallas_call(
        body,
        grid=(e // bm,),
        in_specs=[
            pl.BlockSpec((bm, 16), lambda i: (i, 0)),
            pl.BlockSpec((80, H), lambda i: (0, 0)),
            vec(), vec(), vec(),
            pl.BlockSpec((H, H), lambda i: (0, 0)),
            vec(), vec(), vec(),
        ],
        out_specs=[
            pl.BlockSpec((bm, H), lambda i: (i, 0)),
            pl.BlockSpec((bm, 16), lambda i: (i, 0)),
            pl.BlockSpec((bm, 1), lambda i: (i, 0)),
        ],
        out_shape=[
            jax.ShapeDtypeStruct((e, H), F32),
            jax.ShapeDtypeStruct((e, 16), F32),
            jax.ShapeDtypeStruct((e, 1), F32),
        ],
    )(
        r16,
        e_w1, e_b1.reshape(1, H), e_ln1[0].reshape(1, H), e_ln1[1].reshape(1, H),
        e_w2, e_b2.reshape(1, H), e_ln2[0].reshape(1, H), e_ln2[1].reshape(1, H),
    )


def _tc_angle_embed(qs, qd, a_w1, a_b1, a_ln1, a_w2, a_b2, a_ln2):
    lg = qs.shape[0]
    bm = 2000

    def body(qs_ref, qd_ref, w1, b1, g1, be1, w2, b2, g2, be2, z_ref):
        a = qs_ref[...]
        b = qd_ref[...]
        ci = lax.broadcasted_iota(jnp.int32, (bm, 16), 1)
        prod = jnp.where(ci < 3, a * b, 0.0)
        d3 = jnp.sum(prod, axis=1, keepdims=True)
        l1 = a[:, 3:4]
        l2 = b[:, 3:4]
        cos = -d3 / (l1 * l2 + 1e-8)
        z_ref[...] = _mlp_block(
            _rbf_block(cos, -1.0, 1.0, 40),
            w1[...], b1[...], g1[...], be1[...], w2[...], b2[...], g2[...], be2[...],
        )

    vec = lambda: pl.BlockSpec((1, H), lambda i: (0, 0))
    return pl.pallas_call(
        body,
        grid=(lg // bm,),
        in_specs=[
            pl.BlockSpec((bm, 16), lambda i: (i, 0)),
            pl.BlockSpec((bm, 16), lambda i: (i, 0)),
            pl.BlockSpec((40, H), lambda i: (0, 0)),
            vec(), vec(), vec(),
            pl.BlockSpec((H, H), lambda i: (0, 0)),
            vec(), vec(), vec(),
        ],
        out_specs=pl.BlockSpec((bm, H), lambda i: (i, 0)),
        out_shape=jax.ShapeDtypeStruct((lg, H), F32),
    )(
        qs, qd,
        a_w1, a_b1.reshape(1, H), a_ln1[0].reshape(1, H), a_ln1[1].reshape(1, H),
        a_w2, a_b2.reshape(1, H), a_ln2[0].reshape(1, H), a_ln2[1].reshape(1, H),
    )


def _tc_edge_conv(ga, gb, y, cut, w, b, ln, has_cutoff):
    e = ga.shape[0]
    bm = 2000
    bsum = (b[0] + b[1] + b[2]).reshape(1, H)
    b3 = b[3].reshape(1, H)

    def body(*refs):
        if has_cutoff:
            (ga_r, gb_r, y_r, c_r, w0, w1, w2, w3, bs, b3r, g_r, be_r,
             p_ref, yn_ref) = refs
        else:
            (ga_r, gb_r, y_r, w0, w1, w2, w3, bs, b3r, g_r, be_r,
             p_ref, yn_ref) = refs
        av = ga_r[...]
        ev = (
            jnp.dot(av, w0[...], preferred_element_type=F32)
            + jnp.dot(gb_r[...], w1[...], preferred_element_type=F32)
            + jnp.dot(y_r[...], w2[...], preferred_element_type=F32)
            + bs[...]
        )
        sig = jax.nn.sigmoid(ev)
        if has_cutoff:
            sig = sig * c_r[...]
        bh = jnp.dot(av, w3[...], preferred_element_type=F32) + b3r[...]
        p_ref[:, :H] = sig * bh
        p_ref[:, H:] = sig
        yn_ref[...] = y_r[...] + _silu(_ln(ev, g_r[...], be_r[...]))

    blk = lambda: pl.BlockSpec((bm, H), lambda i: (i, 0))
    mat = lambda: pl.BlockSpec((H, H), lambda i: (0, 0))
    vec = lambda: pl.BlockSpec((1, H), lambda i: (0, 0))
    in_specs = [blk(), blk(), blk()]
    args = [ga, gb, y]
    if has_cutoff:
        in_specs.append(pl.BlockSpec((bm, 1), lambda i: (i, 0)))
        args.append(cut)
    in_specs += [mat(), mat(), mat(), mat(), vec(), vec(), vec(), vec()]
    args += [w[0], w[1], w[2], w[3], bsum, b3,
             ln[2].reshape(1, H), ln[3].reshape(1, H)]
    return pl.pallas_call(
        body,
        grid=(e // bm,),
        in_specs=in_specs,
        out_specs=[
            pl.BlockSpec((bm, 2 * H), lambda i: (i, 0)),
            pl.BlockSpec((bm, H), lambda i: (i, 0)),
        ],
        out_shape=[
            jax.ShapeDtypeStruct((e, 2 * H), F32),
            jax.ShapeDtypeStruct((e, H), F32),
        ],
    )(*args)


def _tc_update(s, x, w4, b4, g, be):
    n = x.shape[0]
    bm = 1000 if n <= 10000 else 2000

    def body(s_ref, x_ref, w_r, b_r, g_r, be_r, o_ref):
        sv = s_ref[...]
        h = sv[:, :H] / (sv[:, H:] + 1e-6)
        xv = x_ref[...]
        pre = jnp.dot(xv, w_r[...], preferred_element_type=F32) + b_r[...] + h
        o_ref[...] = xv + _silu(_ln(pre, g_r[...], be_r[...]))

    return pl.pallas_call(
        body,
        grid=(n // bm,),
        in_specs=[
            pl.BlockSpec((bm, 2 * H), lambda i: (i, 0)),
            pl.BlockSpec((bm, H), lambda i: (i, 0)),
            pl.BlockSpec((H, H), lambda i: (0, 0)),
            pl.BlockSpec((1, H), lambda i: (0, 0)),
            pl.BlockSpec((1, H), lambda i: (0, 0)),
            pl.BlockSpec((1, H), lambda i: (0, 0)),
        ],
        out_specs=pl.BlockSpec((bm, H), lambda i: (i, 0)),
        out_shape=jax.ShapeDtypeStruct((n, H), F32),
    )(s, x, w4, b4.reshape(1, H), g.reshape(1, H), be.reshape(1, H))


def _tc_final(x, fc_w, fc_b):
    n = x.shape[0]

    def body(x_ref, w_r, b_r, o_ref):
        m = jnp.mean(x_ref[...], axis=0, keepdims=True)
        o_ref[...] = jnp.dot(m, w_r[...], preferred_element_type=F32) + b_r[...]

    out = pl.pallas_call(
        body,
        grid=(1,),
        in_specs=[
            pl.BlockSpec((n, H), lambda i: (0, 0)),
            pl.BlockSpec((H, 1), lambda i: (0, 0)),
            pl.BlockSpec((1, 1), lambda i: (0, 0)),
        ],
        out_specs=pl.BlockSpec((1, 1), lambda i: (0, 0)),
        out_shape=jax.ShapeDtypeStruct((1, 1), F32),
    )(x, fc_w, fc_b.reshape(1, 1))
    return out[0, 0]


def _conv(x, y, src, dst, n_seg, w, b, ln, cut):
    ga, gb = _sc_gather2(x, src, dst)
    p, y_new = _tc_edge_conv(ga, gb, y, cut, w, b, ln, cut is not None)
    s = _sc_segsum(p, dst, n_seg)
    x_new = _tc_update(s, x, w[4], b[4], ln[0], ln[1])
    return x_new, y_new


def kernel(r, atom_emb, e_W1, e_b1, e_ln1, e_W2, e_b2, e_ln2, a_W1, a_b1,
           a_ln1, a_W2, a_b2, a_ln2, conv_W, conv_b, conv_ln, fc_W, fc_b,
           atomic_number, edge_index, lg_edge_index):
    n_nodes = atomic_number.shape[0]
    n_edges = r.shape[0]
    src, dst = edge_index[0], edge_index[1]
    lsrc, ldst = lg_edge_index[0], lg_edge_index[1]

    r16 = jnp.pad(r.astype(F32), ((0, 0), (0, 13)))
    x = _tc_atom_embed(atomic_number.astype(jnp.int32), atom_emb)
    y, q, cut = _tc_edge_embed(r16, e_W1, e_b1, e_ln1, e_W2, e_b2, e_ln2)
    qs, qd = _sc_gather2(q, lsrc, ldst)
    z = _tc_angle_embed(qs, qd, a_W1, a_b1, a_ln1, a_W2, a_b2, a_ln2)

    li = 0
    for _ in range(2):
        x, m = _conv(x, y, src, dst, n_nodes, conv_W[li], conv_b[li],
                     conv_ln[li], cut)
        li += 1
        y, z = _conv(m, z, lsrc, ldst, n_edges, conv_W[li], conv_b[li],
                     conv_ln[li], None)
        li += 1
    for _ in range(2):
        x, y = _conv(x, y, src, dst, n_nodes, conv_W[li], conv_b[li],
                     conv_ln[li], cut)
        li += 1
    return _tc_final(x, fc_W, fc_b)
